# Initial kernel scaffold; baseline (speedup 1.0000x reference)
import jax
import jax.numpy as jnp
from jax.experimental import pallas as pl


def kernel(points, xyz2, feat2, xyz3, feat3, xyz4, feat4, W_fc, W_cls):
    raise NotImplementedError("write your pallas kernel here")


# speedup vs baseline: 13.3858x; 13.3858x over previous
import functools

import jax
import jax.numpy as jnp
from jax.experimental import pallas as pl
from jax.experimental.pallas import tpu as pltpu
from jax.experimental.pallas import tpu_sc as plsc

QB = 128
IB = 1024
GW = 128


def _top3(d, m):
    iota = jax.lax.broadcasted_iota(jnp.int32, d.shape, 1)
    dists = []
    idxs = []
    for k in range(3):
        mval = jnp.min(d, axis=1)
        eq = d == mval[:, None]
        fi = jnp.min(jnp.where(eq, iota, m), axis=1)
        dists.append(mval)
        idxs.append(fi)
        if k < 2:
            sel = iota == fi[:, None]
            d = jnp.where(sel, jnp.float32(jnp.inf), d)
    return dists, idxs


def _nn_kernel(qt_ref, kt_ref, d2_ref, i2_ref, d3_ref, i3_ref, d4_ref,
               i4_ref, *, sizes):
    qt = qt_ref[...]
    kt = kt_ref[...]
    qq = jnp.sum(qt * qt, axis=0)
    kk = jnp.sum(kt * kt, axis=0)
    dot = jnp.dot(qt.T, kt, preferred_element_type=jnp.float32)
    d = qq[:, None] + kk[None, :] - 2.0 * dot
    off = 0
    for (d_ref, i_ref), m in zip(((d2_ref, i2_ref), (d3_ref, i3_ref),
                                  (d4_ref, i4_ref)), sizes):
        dists, idxs = _top3(d[:, off:off + m], m)
        for k in range(3):
            d_ref[k, :] = dists[k]
            i_ref[k, :] = idxs[k]
        off += m


def _gather_kernel(f2_hbm, i2_hbm, f3_hbm, i3_hbm, f4_hbm, i4_hbm,
                   g2_hbm, g3_hbm, g4_hbm):
    for f_hbm, i_hbm, g_hbm in ((f2_hbm, i2_hbm, g2_hbm),
                                (f3_hbm, i3_hbm, g3_hbm),
                                (f4_hbm, i4_hbm, g4_hbm)):
        def body(i_vmem, o_vmem, f_hbm=f_hbm):
            pltpu.sync_copy(f_hbm.at[i_vmem.at[0]], o_vmem)

        n_idx = i_hbm.shape[1]
        pltpu.emit_pipeline(
            body,
            grid=(n_idx // GW,),
            in_specs=[pl.BlockSpec((1, GW), index_map=lambda i: (0, i))],
            out_specs=[pl.BlockSpec((GW, f_hbm.shape[1]),
                                    index_map=lambda i: (i, 0))],
            core_axis_name=("core", "subcore"),
            dimension_semantics=(pltpu.PARALLEL,),
        )(i_hbm, g_hbm)


def _interp_kernel(d2_ref, d3_ref, d4_ref, g2_ref, g3_ref, g4_ref,
                   wfc_ref, bcls_ref, out_ref, *, cs):
    ps = []
    for (d_ref, g_ref), c in zip(((d2_ref, g2_ref), (d3_ref, g3_ref),
                                  (d4_ref, g4_ref)), cs):
        dr0 = 1.0 / (d_ref[0, :] + 1e-08)
        dr1 = 1.0 / (d_ref[1, :] + 1e-08)
        dr2 = 1.0 / (d_ref[2, :] + 1e-08)
        norm = jnp.sum(jnp.stack([dr0, dr1, dr2], axis=0), axis=0)
        w = jnp.stack([dr0 / norm, dr1 / norm, dr2 / norm], axis=1)
        gath = jnp.stack([g_ref[0][:, :c], g_ref[1][:, :c],
                          g_ref[2][:, :c]], axis=1)
        ps.append(jnp.sum(gath * w[:, :, None], axis=1))
    pf = jnp.concatenate(ps, axis=1)
    pw = jnp.dot(pf, wfc_ref[...].T, preferred_element_type=jnp.float32)
    out = jnp.dot(pw, bcls_ref[...], preferred_element_type=jnp.float32)
    out_ref[...] = out[:, 0:1]


def kernel(points, xyz2, feat2, xyz3, feat3, xyz4, feat4, W_fc, W_cls):
    n = points.shape[0]
    m2, m3, m4 = xyz2.shape[0], xyz3.shape[0], xyz4.shape[0]
    c2, c3, c4 = feat2.shape[1], feat3.shape[1], feat4.shape[1]
    mt = m2 + m3 + m4
    nb = n // QB

    qt = points.T
    kt = jnp.concatenate([xyz2, xyz3, xyz4], axis=0).T

    d_spec = pl.BlockSpec((3, QB), lambda i: (0, i))
    d2, i2, d3, i3, d4, i4 = pl.pallas_call(
        functools.partial(_nn_kernel, sizes=(m2, m3, m4)),
        grid=(nb,),
        in_specs=[
            pl.BlockSpec((3, QB), lambda i: (0, i)),
            pl.BlockSpec((3, mt), lambda i: (0, 0)),
        ],
        out_specs=[d_spec] * 6,
        out_shape=[
            jax.ShapeDtypeStruct((3, n), jnp.float32),
            jax.ShapeDtypeStruct((3, n), jnp.int32),
        ] * 3,
        compiler_params=pltpu.CompilerParams(
            dimension_semantics=("arbitrary",),
        ),
    )(qt, kt)

    c2p = 128
    feat2p = jnp.pad(feat2, ((0, 0), (0, c2p - c2)))

    sc_gather = pl.kernel(
        _gather_kernel,
        out_type=[
            jax.ShapeDtypeStruct((3 * n, c2p), jnp.float32),
            jax.ShapeDtypeStruct((3 * n, c3), jnp.float32),
            jax.ShapeDtypeStruct((3 * n, c4), jnp.float32),
        ],
        mesh=plsc.VectorSubcoreMesh(core_axis_name="core",
                                    subcore_axis_name="subcore"),
    )
    g2, g3, g4 = sc_gather(feat2p, i2.reshape(1, 3 * n),
                           feat3, i3.reshape(1, 3 * n),
                           feat4, i4.reshape(1, 3 * n))

    bcls = jnp.pad(W_cls.T, ((0, 0), (0, 127)))

    out = pl.pallas_call(
        functools.partial(_interp_kernel, cs=(c2, c3, c4)),
        grid=(n // IB,),
        in_specs=[
            pl.BlockSpec((3, IB), lambda i: (0, i)),
            pl.BlockSpec((3, IB), lambda i: (0, i)),
            pl.BlockSpec((3, IB), lambda i: (0, i)),
            pl.BlockSpec((3, IB, c2p), lambda i: (0, i, 0)),
            pl.BlockSpec((3, IB, c3), lambda i: (0, i, 0)),
            pl.BlockSpec((3, IB, c4), lambda i: (0, i, 0)),
            pl.BlockSpec((64, 320), lambda i: (0, 0)),
            pl.BlockSpec((64, 128), lambda i: (0, 0)),
        ],
        out_specs=pl.BlockSpec((IB, 1), lambda i: (i, 0)),
        out_shape=jax.ShapeDtypeStruct((n, 1), jnp.float32),
        compiler_params=pltpu.CompilerParams(
            dimension_semantics=("arbitrary",),
        ),
    )(d2, d3, d4, g2.reshape(3, n, c2p), g3.reshape(3, n, c3),
      g4.reshape(3, n, c4), W_fc, bcls)

    return out


# speedup vs baseline: 14.0864x; 1.0523x over previous
import functools

import jax
import jax.numpy as jnp
from jax.experimental import pallas as pl
from jax.experimental.pallas import tpu as pltpu
from jax.experimental.pallas import tpu_sc as plsc

QB = 128
IB = 1024
GW = 128


def _top3(d_tiles, m):
    qb = d_tiles[0].shape[0]
    inf = jnp.full((qb, 128), jnp.inf, jnp.float32)
    zero = jnp.zeros((qb, 128), jnp.int32)
    b1, b2, b3 = inf, inf, inf
    t1, t2, t3 = zero, zero, zero
    for t, v in enumerate(d_tiles):
        c1 = v < b1
        c2 = v < b2
        c3 = v < b3
        nb3 = jnp.where(c2, b2, jnp.where(c3, v, b3))
        nt3 = jnp.where(c2, t2, jnp.where(c3, t, t3))
        nb2 = jnp.where(c1, b1, jnp.where(c2, v, b2))
        nt2 = jnp.where(c1, t1, jnp.where(c2, t, t2))
        b1 = jnp.where(c1, v, b1)
        t1 = jnp.where(c1, t, t1)
        b2, b3, t2, t3 = nb2, nb3, nt2, nt3
    lane = jax.lax.broadcasted_iota(jnp.int32, (qb, 128), 1)
    vals = jnp.concatenate([b1, b2, b3], axis=1)
    idx = jnp.concatenate([t1 * 128 + lane, t2 * 128 + lane,
                           t3 * 128 + lane], axis=1)
    dists = []
    idxs = []
    for k in range(3):
        mv = jnp.min(vals, axis=1)
        hit = vals == mv[:, None]
        fi = jnp.min(jnp.where(hit, idx, m), axis=1)
        dists.append(mv)
        idxs.append(fi)
        if k < 2:
            vals = jnp.where(hit & (idx == fi[:, None]),
                             jnp.float32(jnp.inf), vals)
    return dists, idxs


def _nn_kernel(qt_ref, kt_ref, d2_ref, i2_ref, d3_ref, i3_ref, d4_ref,
               i4_ref, *, sizes):
    qt = qt_ref[...]
    kt = kt_ref[...]
    qq = jnp.sum(qt * qt, axis=0)
    kk = jnp.sum(kt * kt, axis=0)
    dot = jnp.dot(qt.T, kt, preferred_element_type=jnp.float32)
    off = 0
    for (d_ref, i_ref), m in zip(((d2_ref, i2_ref), (d3_ref, i3_ref),
                                  (d4_ref, i4_ref)), sizes):
        tiles = [
            qq[:, None] + kk[None, off + t * 128:off + (t + 1) * 128]
            - 2.0 * dot[:, off + t * 128:off + (t + 1) * 128]
            for t in range(m // 128)
        ]
        dists, idxs = _top3(tiles, m)
        for k in range(3):
            d_ref[k, :] = dists[k]
            i_ref[k, :] = idxs[k]
        off += m


def _gather_kernel(f2_hbm, i2_hbm, f3_hbm, i3_hbm, f4_hbm, i4_hbm,
                   g2_hbm, g3_hbm, g4_hbm):
    for f_hbm, i_hbm, g_hbm in ((f2_hbm, i2_hbm, g2_hbm),
                                (f3_hbm, i3_hbm, g3_hbm),
                                (f4_hbm, i4_hbm, g4_hbm)):
        def body(i_vmem, o_vmem, f_hbm=f_hbm):
            pltpu.sync_copy(f_hbm.at[i_vmem.at[0]], o_vmem)

        n_idx = i_hbm.shape[1]
        pltpu.emit_pipeline(
            body,
            grid=(n_idx // GW,),
            in_specs=[pl.BlockSpec((1, GW), index_map=lambda i: (0, i))],
            out_specs=[pl.BlockSpec((GW, f_hbm.shape[1]),
                                    index_map=lambda i: (i, 0))],
            core_axis_name=("core", "subcore"),
            dimension_semantics=(pltpu.PARALLEL,),
        )(i_hbm, g_hbm)


def _interp_kernel(d2_ref, d3_ref, d4_ref, g2_ref, g3_ref, g4_ref,
                   wfc_ref, bcls_ref, out_ref, *, cs):
    ps = []
    for (d_ref, g_ref), c in zip(((d2_ref, g2_ref), (d3_ref, g3_ref),
                                  (d4_ref, g4_ref)), cs):
        dr0 = 1.0 / (d_ref[0, :] + 1e-08)
        dr1 = 1.0 / (d_ref[1, :] + 1e-08)
        dr2 = 1.0 / (d_ref[2, :] + 1e-08)
        norm = jnp.sum(jnp.stack([dr0, dr1, dr2], axis=0), axis=0)
        w = jnp.stack([dr0 / norm, dr1 / norm, dr2 / norm], axis=1)
        gath = jnp.stack([g_ref[0][:, :c], g_ref[1][:, :c],
                          g_ref[2][:, :c]], axis=1)
        ps.append(jnp.sum(gath * w[:, :, None], axis=1))
    pf = jnp.concatenate(ps, axis=1)
    pw = jnp.dot(pf, wfc_ref[...].T, preferred_element_type=jnp.float32)
    out = jnp.dot(pw, bcls_ref[...], preferred_element_type=jnp.float32)
    out_ref[...] = out[:, 0:1]


def kernel(points, xyz2, feat2, xyz3, feat3, xyz4, feat4, W_fc, W_cls):
    n = points.shape[0]
    m2, m3, m4 = xyz2.shape[0], xyz3.shape[0], xyz4.shape[0]
    c2, c3, c4 = feat2.shape[1], feat3.shape[1], feat4.shape[1]
    mt = m2 + m3 + m4
    nb = n // QB

    qt = points.T
    kt = jnp.concatenate([xyz2, xyz3, xyz4], axis=0).T

    d_spec = pl.BlockSpec((3, QB), lambda i: (0, i))
    d2, i2, d3, i3, d4, i4 = pl.pallas_call(
        functools.partial(_nn_kernel, sizes=(m2, m3, m4)),
        grid=(nb,),
        in_specs=[
            pl.BlockSpec((3, QB), lambda i: (0, i)),
            pl.BlockSpec((3, mt), lambda i: (0, 0)),
        ],
        out_specs=[d_spec] * 6,
        out_shape=[
            jax.ShapeDtypeStruct((3, n), jnp.float32),
            jax.ShapeDtypeStruct((3, n), jnp.int32),
        ] * 3,
        compiler_params=pltpu.CompilerParams(
            dimension_semantics=("arbitrary",),
        ),
    )(qt, kt)

    c2p = 128
    feat2p = jnp.pad(feat2, ((0, 0), (0, c2p - c2)))

    sc_gather = pl.kernel(
        _gather_kernel,
        out_type=[
            jax.ShapeDtypeStruct((3 * n, c2p), jnp.float32),
            jax.ShapeDtypeStruct((3 * n, c3), jnp.float32),
            jax.ShapeDtypeStruct((3 * n, c4), jnp.float32),
        ],
        mesh=plsc.VectorSubcoreMesh(core_axis_name="core",
                                    subcore_axis_name="subcore"),
    )
    g2, g3, g4 = sc_gather(feat2p, i2.reshape(1, 3 * n),
                           feat3, i3.reshape(1, 3 * n),
                           feat4, i4.reshape(1, 3 * n))

    bcls = jnp.pad(W_cls.T, ((0, 0), (0, 127)))

    out = pl.pallas_call(
        functools.partial(_interp_kernel, cs=(c2, c3, c4)),
        grid=(n // IB,),
        in_specs=[
            pl.BlockSpec((3, IB), lambda i: (0, i)),
            pl.BlockSpec((3, IB), lambda i: (0, i)),
            pl.BlockSpec((3, IB), lambda i: (0, i)),
            pl.BlockSpec((3, IB, c2p), lambda i: (0, i, 0)),
            pl.BlockSpec((3, IB, c3), lambda i: (0, i, 0)),
            pl.BlockSpec((3, IB, c4), lambda i: (0, i, 0)),
            pl.BlockSpec((64, 320), lambda i: (0, 0)),
            pl.BlockSpec((64, 128), lambda i: (0, 0)),
        ],
        out_specs=pl.BlockSpec((IB, 1), lambda i: (i, 0)),
        out_shape=jax.ShapeDtypeStruct((n, 1), jnp.float32),
        compiler_params=pltpu.CompilerParams(
            dimension_semantics=("arbitrary",),
        ),
    )(d2, d3, d4, g2.reshape(3, n, c2p), g3.reshape(3, n, c3),
      g4.reshape(3, n, c4), W_fc, bcls)

    return out


# speedup vs baseline: 14.0877x; 1.0001x over previous
import functools

import jax
import jax.numpy as jnp
from jax.experimental import pallas as pl
from jax.experimental.pallas import tpu as pltpu
from jax.experimental.pallas import tpu_sc as plsc

QB = 128
IB = 1024
GW = 128


def _top3(d_tiles, m):
    qb = d_tiles[0].shape[0]
    inf = jnp.full((qb, 128), jnp.inf, jnp.float32)
    zero = jnp.zeros((qb, 128), jnp.int32)
    b1, b2, b3 = inf, inf, inf
    t1, t2, t3 = zero, zero, zero
    for t, v in enumerate(d_tiles):
        c1 = v < b1
        c2 = v < b2
        c3 = v < b3
        nb3 = jnp.where(c2, b2, jnp.where(c3, v, b3))
        nt3 = jnp.where(c2, t2, jnp.where(c3, t, t3))
        nb2 = jnp.where(c1, b1, jnp.where(c2, v, b2))
        nt2 = jnp.where(c1, t1, jnp.where(c2, t, t2))
        b1 = jnp.where(c1, v, b1)
        t1 = jnp.where(c1, t, t1)
        b2, b3, t2, t3 = nb2, nb3, nt2, nt3
    lane = jax.lax.broadcasted_iota(jnp.int32, (qb, 128), 1)
    vals = jnp.concatenate([b1, b2, b3], axis=1)
    idx = jnp.concatenate([t1 * 128 + lane, t2 * 128 + lane,
                           t3 * 128 + lane], axis=1)
    dists = []
    idxs = []
    for k in range(3):
        mv = jnp.min(vals, axis=1)
        hit = vals == mv[:, None]
        fi = jnp.min(jnp.where(hit, idx, m), axis=1)
        dists.append(mv)
        idxs.append(fi)
        if k < 2:
            vals = jnp.where(hit & (idx == fi[:, None]),
                             jnp.float32(jnp.inf), vals)
    return dists, idxs


def _nn_kernel(qt_ref, kt_ref, d2_ref, i2_ref, d3_ref, i3_ref, d4_ref,
               i4_ref, *, sizes):
    qt = qt_ref[...]
    kt = kt_ref[...]
    qq = jnp.sum(qt * qt, axis=0)
    kk = jnp.sum(kt * kt, axis=0)
    dot = jnp.dot(qt.T, kt, preferred_element_type=jnp.float32)
    off = 0
    for (d_ref, i_ref), m in zip(((d2_ref, i2_ref), (d3_ref, i3_ref),
                                  (d4_ref, i4_ref)), sizes):
        tiles = [
            qq[:, None] + kk[None, off + t * 128:off + (t + 1) * 128]
            - 2.0 * dot[:, off + t * 128:off + (t + 1) * 128]
            for t in range(m // 128)
        ]
        dists, idxs = _top3(tiles, m)
        for k in range(3):
            d_ref[k, :] = dists[k]
            i_ref[k, :] = idxs[k]
        off += m


def _gather_kernel(f2_hbm, i2_hbm, f3_hbm, i3_hbm, f4_hbm, i4_hbm,
                   g2_hbm, g3_hbm, g4_hbm):
    for f_hbm, i_hbm, g_hbm in ((f2_hbm, i2_hbm, g2_hbm),
                                (f3_hbm, i3_hbm, g3_hbm),
                                (f4_hbm, i4_hbm, g4_hbm)):
        def body(i_vmem, o_vmem, f_hbm=f_hbm):
            pltpu.sync_copy(f_hbm.at[i_vmem.at[0]], o_vmem)

        n_idx = i_hbm.shape[1]
        pltpu.emit_pipeline(
            body,
            grid=(n_idx // GW,),
            in_specs=[pl.BlockSpec((1, GW), index_map=lambda i: (0, i))],
            out_specs=[pl.BlockSpec((GW, f_hbm.shape[1]),
                                    index_map=lambda i: (i, 0))],
            core_axis_name=("core", "subcore"),
            dimension_semantics=(pltpu.PARALLEL,),
        )(i_hbm, g_hbm)


def _interp_kernel(d2_ref, d3_ref, d4_ref, g2_ref, g3_ref, g4_ref,
                   wfc_ref, bcls_ref, out_ref, *, cs):
    ps = []
    for (d_ref, g_ref), c in zip(((d2_ref, g2_ref), (d3_ref, g3_ref),
                                  (d4_ref, g4_ref)), cs):
        dr0 = 1.0 / (d_ref[0, :] + 1e-08)
        dr1 = 1.0 / (d_ref[1, :] + 1e-08)
        dr2 = 1.0 / (d_ref[2, :] + 1e-08)
        norm = jnp.sum(jnp.stack([dr0, dr1, dr2], axis=0), axis=0)
        w = jnp.stack([dr0 / norm, dr1 / norm, dr2 / norm], axis=1)
        gath = jnp.stack([g_ref[0][:, :c], g_ref[1][:, :c],
                          g_ref[2][:, :c]], axis=1)
        ps.append(jnp.sum(gath * w[:, :, None], axis=1))
    pf = jnp.concatenate(ps, axis=1)
    pw = jnp.dot(pf, wfc_ref[...].T, preferred_element_type=jnp.float32)
    out = jnp.dot(pw, bcls_ref[...], preferred_element_type=jnp.float32)
    out_ref[...] = out[:, 0:1]


def kernel(points, xyz2, feat2, xyz3, feat3, xyz4, feat4, W_fc, W_cls):
    n = points.shape[0]
    m2, m3, m4 = xyz2.shape[0], xyz3.shape[0], xyz4.shape[0]
    c2, c3, c4 = feat2.shape[1], feat3.shape[1], feat4.shape[1]
    mt = m2 + m3 + m4
    nb = n // QB

    qt = points.T
    kt = jnp.concatenate([xyz2, xyz3, xyz4], axis=0).T

    d_spec = pl.BlockSpec((3, QB), lambda i: (0, i))
    d2, i2, d3, i3, d4, i4 = pl.pallas_call(
        functools.partial(_nn_kernel, sizes=(m2, m3, m4)),
        grid=(nb,),
        in_specs=[
            pl.BlockSpec((3, QB), lambda i: (0, i)),
            pl.BlockSpec((3, mt), lambda i: (0, 0)),
        ],
        out_specs=[d_spec] * 6,
        out_shape=[
            jax.ShapeDtypeStruct((3, n), jnp.float32),
            jax.ShapeDtypeStruct((3, n), jnp.int32),
        ] * 3,
        compiler_params=pltpu.CompilerParams(
            dimension_semantics=("parallel",),
        ),
    )(qt, kt)

    c2p = 128
    feat2p = jnp.pad(feat2, ((0, 0), (0, c2p - c2)))

    sc_gather = pl.kernel(
        _gather_kernel,
        out_type=[
            jax.ShapeDtypeStruct((3 * n, c2p), jnp.float32),
            jax.ShapeDtypeStruct((3 * n, c3), jnp.float32),
            jax.ShapeDtypeStruct((3 * n, c4), jnp.float32),
        ],
        mesh=plsc.VectorSubcoreMesh(core_axis_name="core",
                                    subcore_axis_name="subcore"),
    )
    g2, g3, g4 = sc_gather(feat2p, i2.reshape(1, 3 * n),
                           feat3, i3.reshape(1, 3 * n),
                           feat4, i4.reshape(1, 3 * n))

    bcls = jnp.pad(W_cls.T, ((0, 0), (0, 127)))

    out = pl.pallas_call(
        functools.partial(_interp_kernel, cs=(c2, c3, c4)),
        grid=(n // IB,),
        in_specs=[
            pl.BlockSpec((3, IB), lambda i: (0, i)),
            pl.BlockSpec((3, IB), lambda i: (0, i)),
            pl.BlockSpec((3, IB), lambda i: (0, i)),
            pl.BlockSpec((3, IB, c2p), lambda i: (0, i, 0)),
            pl.BlockSpec((3, IB, c3), lambda i: (0, i, 0)),
            pl.BlockSpec((3, IB, c4), lambda i: (0, i, 0)),
            pl.BlockSpec((64, 320), lambda i: (0, 0)),
            pl.BlockSpec((64, 128), lambda i: (0, 0)),
        ],
        out_specs=pl.BlockSpec((IB, 1), lambda i: (i, 0)),
        out_shape=jax.ShapeDtypeStruct((n, 1), jnp.float32),
        compiler_params=pltpu.CompilerParams(
            dimension_semantics=("parallel",),
        ),
    )(d2, d3, d4, g2.reshape(3, n, c2p), g3.reshape(3, n, c3),
      g4.reshape(3, n, c4), W_fc, bcls)

    return out


# speedup vs baseline: 14.8068x; 1.0510x over previous
import functools

import jax
import jax.numpy as jnp
from jax.experimental import pallas as pl
from jax.experimental.pallas import tpu as pltpu
from jax.experimental.pallas import tpu_sc as plsc

QB = 128
IB = 1024
GW = 128


def _top3(d_tiles, m):
    qb = d_tiles[0].shape[0]
    inf = jnp.full((qb, 128), jnp.inf, jnp.float32)
    zero = jnp.zeros((qb, 128), jnp.int32)
    b1, b2, b3 = inf, inf, inf
    t1, t2, t3 = zero, zero, zero
    for t, v in enumerate(d_tiles):
        c1 = v < b1
        c2 = v < b2
        c3 = v < b3
        nb3 = jnp.where(c2, b2, jnp.where(c3, v, b3))
        nt3 = jnp.where(c2, t2, jnp.where(c3, t, t3))
        nb2 = jnp.where(c1, b1, jnp.where(c2, v, b2))
        nt2 = jnp.where(c1, t1, jnp.where(c2, t, t2))
        b1 = jnp.where(c1, v, b1)
        t1 = jnp.where(c1, t, t1)
        b2, b3, t2, t3 = nb2, nb3, nt2, nt3
    lane = jax.lax.broadcasted_iota(jnp.int32, (qb, 128), 1)
    vals = jnp.concatenate([b1, b2, b3], axis=1)
    idx = jnp.concatenate([t1 * 128 + lane, t2 * 128 + lane,
                           t3 * 128 + lane], axis=1)
    dists = []
    idxs = []
    for k in range(3):
        mv = jnp.min(vals, axis=1)
        hit = vals == mv[:, None]
        fi = jnp.min(jnp.where(hit, idx, m), axis=1)
        dists.append(mv)
        idxs.append(fi)
        if k < 2:
            vals = jnp.where(hit & (idx == fi[:, None]),
                             jnp.float32(jnp.inf), vals)
    return dists, idxs


def _nn_kernel(qt_ref, kt_ref, d2_ref, i2_ref, d3_ref, i3_ref, d4_ref,
               i4_ref, *, sizes):
    qt = qt_ref[...]
    kt = kt_ref[...]
    qq = jnp.sum(qt * qt, axis=0)
    kk = jnp.sum(kt * kt, axis=0)
    dot = jnp.dot(qt.T, kt, preferred_element_type=jnp.float32)
    off = 0
    for (d_ref, i_ref), m in zip(((d2_ref, i2_ref), (d3_ref, i3_ref),
                                  (d4_ref, i4_ref)), sizes):
        tiles = [
            qq[:, None] + kk[None, off + t * 128:off + (t + 1) * 128]
            - 2.0 * dot[:, off + t * 128:off + (t + 1) * 128]
            for t in range(m // 128)
        ]
        dists, idxs = _top3(tiles, m)
        for k in range(3):
            d_ref[k, :] = dists[k]
            i_ref[k, :] = idxs[k]
        off += m


def _gather_kernel(f2_hbm, i2_hbm, f3_hbm, i3_hbm, f4_hbm, i4_hbm,
                   g2_hbm, g3_hbm, g4_hbm):
    for f_hbm, i_hbm, g_hbm in ((f2_hbm, i2_hbm, g2_hbm),
                                (f3_hbm, i3_hbm, g3_hbm),
                                (f4_hbm, i4_hbm, g4_hbm)):
        def body(i_vmem, o_vmem, f_hbm=f_hbm):
            pltpu.sync_copy(f_hbm.at[i_vmem.at[0]], o_vmem)

        n_idx = i_hbm.shape[1]
        pltpu.emit_pipeline(
            body,
            grid=(n_idx // GW,),
            in_specs=[pl.BlockSpec((1, GW), index_map=lambda i: (0, i))],
            out_specs=[pl.BlockSpec((GW, f_hbm.shape[1]),
                                    index_map=lambda i: (i, 0))],
            core_axis_name=("core", "subcore"),
            dimension_semantics=(pltpu.PARALLEL,),
        )(i_hbm, g_hbm)


def _interp_kernel(d2_ref, d3_ref, d4_ref, g2_ref, g3_ref, g4_ref,
                   wfc_ref, bcls_ref, out_ref, *, cs):
    ps = []
    for (d_ref, g_ref), c in zip(((d2_ref, g2_ref), (d3_ref, g3_ref),
                                  (d4_ref, g4_ref)), cs):
        dr0 = 1.0 / (d_ref[0, :] + 1e-08)
        dr1 = 1.0 / (d_ref[1, :] + 1e-08)
        dr2 = 1.0 / (d_ref[2, :] + 1e-08)
        norm = jnp.sum(jnp.stack([dr0, dr1, dr2], axis=0), axis=0)
        w = jnp.stack([dr0 / norm, dr1 / norm, dr2 / norm], axis=1)
        gath = jnp.stack([g_ref[0][:, :c], g_ref[1][:, :c],
                          g_ref[2][:, :c]], axis=1)
        ps.append(jnp.sum(gath * w[:, :, None], axis=1))
    pf = jnp.concatenate(ps, axis=1)
    pw = jnp.dot(pf, wfc_ref[...].T, preferred_element_type=jnp.float32)
    out = jnp.dot(pw, bcls_ref[...], preferred_element_type=jnp.float32)
    out_ref[...] = out[:, 0:1]


def kernel(points, xyz2, feat2, xyz3, feat3, xyz4, feat4, W_fc, W_cls):
    n = points.shape[0]
    m2, m3, m4 = xyz2.shape[0], xyz3.shape[0], xyz4.shape[0]
    c2, c3, c4 = feat2.shape[1], feat3.shape[1], feat4.shape[1]
    mt = m2 + m3 + m4

    qt = points.T
    kt = jnp.concatenate([xyz2, xyz3, xyz4], axis=0).T

    c2p = 128
    feat2p = jnp.pad(feat2, ((0, 0), (0, c2p - c2)))
    bcls = jnp.pad(W_cls.T, ((0, 0), (0, 127)))

    nh = n // 2
    sc_gather = pl.kernel(
        _gather_kernel,
        out_type=[
            jax.ShapeDtypeStruct((3 * nh, c2p), jnp.float32),
            jax.ShapeDtypeStruct((3 * nh, c3), jnp.float32),
            jax.ShapeDtypeStruct((3 * nh, c4), jnp.float32),
        ],
        mesh=plsc.VectorSubcoreMesh(core_axis_name="core",
                                    subcore_axis_name="subcore"),
    )

    d_spec = pl.BlockSpec((3, QB), lambda i: (0, i))
    outs = []
    for h in range(2):
        qt_h = jax.lax.slice(qt, (0, h * nh), (3, (h + 1) * nh))
        d2, i2, d3, i3, d4, i4 = pl.pallas_call(
            functools.partial(_nn_kernel, sizes=(m2, m3, m4)),
            grid=(nh // QB,),
            in_specs=[
                pl.BlockSpec((3, QB), lambda i: (0, i)),
                pl.BlockSpec((3, mt), lambda i: (0, 0)),
            ],
            out_specs=[d_spec] * 6,
            out_shape=[
                jax.ShapeDtypeStruct((3, nh), jnp.float32),
                jax.ShapeDtypeStruct((3, nh), jnp.int32),
            ] * 3,
            compiler_params=pltpu.CompilerParams(
                dimension_semantics=("parallel",),
            ),
        )(qt_h, kt)

        g2, g3, g4 = sc_gather(feat2p, i2.reshape(1, 3 * nh),
                               feat3, i3.reshape(1, 3 * nh),
                               feat4, i4.reshape(1, 3 * nh))

        out_h = pl.pallas_call(
            functools.partial(_interp_kernel, cs=(c2, c3, c4)),
            grid=(nh // IB,),
            in_specs=[
                pl.BlockSpec((3, IB), lambda i: (0, i)),
                pl.BlockSpec((3, IB), lambda i: (0, i)),
                pl.BlockSpec((3, IB), lambda i: (0, i)),
                pl.BlockSpec((3, IB, c2p), lambda i: (0, i, 0)),
                pl.BlockSpec((3, IB, c3), lambda i: (0, i, 0)),
                pl.BlockSpec((3, IB, c4), lambda i: (0, i, 0)),
                pl.BlockSpec((64, 320), lambda i: (0, 0)),
                pl.BlockSpec((64, 128), lambda i: (0, 0)),
            ],
            out_specs=pl.BlockSpec((IB, 1), lambda i: (i, 0)),
            out_shape=jax.ShapeDtypeStruct((nh, 1), jnp.float32),
            compiler_params=pltpu.CompilerParams(
                dimension_semantics=("parallel",),
            ),
        )(d2, d3, d4, g2.reshape(3, nh, c2p), g3.reshape(3, nh, c3),
          g4.reshape(3, nh, c4), W_fc, bcls)
        outs.append(out_h)

    return jnp.concatenate(outs, axis=0)


# speedup vs baseline: 14.8524x; 1.0031x over previous
import functools

import jax
import jax.numpy as jnp
from jax.experimental import pallas as pl
from jax.experimental.pallas import tpu as pltpu
from jax.experimental.pallas import tpu_sc as plsc

QB = 128
IB = 1024
GW = 128


def _top3(d_tiles, m):
    qb = d_tiles[0].shape[0]
    inf = jnp.full((qb, 128), jnp.inf, jnp.float32)
    zero = jnp.zeros((qb, 128), jnp.int32)
    b1, b2, b3 = inf, inf, inf
    t1, t2, t3 = zero, zero, zero
    for t, v in enumerate(d_tiles):
        c1 = v < b1
        c2 = v < b2
        c3 = v < b3
        nb3 = jnp.where(c2, b2, jnp.where(c3, v, b3))
        nt3 = jnp.where(c2, t2, jnp.where(c3, t, t3))
        nb2 = jnp.where(c1, b1, jnp.where(c2, v, b2))
        nt2 = jnp.where(c1, t1, jnp.where(c2, t, t2))
        b1 = jnp.where(c1, v, b1)
        t1 = jnp.where(c1, t, t1)
        b2, b3, t2, t3 = nb2, nb3, nt2, nt3
    lane = jax.lax.broadcasted_iota(jnp.int32, (qb, 128), 1)
    i1 = t1 * 128 + lane
    i2 = t2 * 128 + lane
    i3 = t3 * 128 + lane
    dists = []
    idxs = []
    for k in range(3):
        mv = jnp.min(b1, axis=1)
        hitv = b1 == mv[:, None]
        fi = jnp.min(jnp.where(hitv, i1, m), axis=1)
        dists.append(mv)
        idxs.append(fi)
        if k < 2:
            hit = hitv & (i1 == fi[:, None])
            b1 = jnp.where(hit, b2, b1)
            i1 = jnp.where(hit, i2, i1)
            b2 = jnp.where(hit, b3, b2)
            i2 = jnp.where(hit, i3, i2)
            b3 = jnp.where(hit, jnp.float32(jnp.inf), b3)
    return dists, idxs


def _nn_kernel(qt_ref, kt_ref, d2_ref, i2_ref, d3_ref, i3_ref, d4_ref,
               i4_ref, *, sizes):
    qt = qt_ref[...]
    kt = kt_ref[...]
    qq = jnp.sum(qt * qt, axis=0)
    kk = jnp.sum(kt * kt, axis=0)
    dot = jnp.dot(qt.T, kt, preferred_element_type=jnp.float32)
    off = 0
    for (d_ref, i_ref), m in zip(((d2_ref, i2_ref), (d3_ref, i3_ref),
                                  (d4_ref, i4_ref)), sizes):
        tiles = [
            qq[:, None] + kk[None, off + t * 128:off + (t + 1) * 128]
            - 2.0 * dot[:, off + t * 128:off + (t + 1) * 128]
            for t in range(m // 128)
        ]
        dists, idxs = _top3(tiles, m)
        for k in range(3):
            d_ref[k, :] = dists[k]
            i_ref[k, :] = idxs[k]
        off += m


def _gather_kernel(f2_hbm, i2_hbm, f3_hbm, i3_hbm, f4_hbm, i4_hbm,
                   g2_hbm, g3_hbm, g4_hbm):
    for f_hbm, i_hbm, g_hbm in ((f2_hbm, i2_hbm, g2_hbm),
                                (f3_hbm, i3_hbm, g3_hbm),
                                (f4_hbm, i4_hbm, g4_hbm)):
        def body(i_vmem, o_vmem, f_hbm=f_hbm):
            pltpu.sync_copy(f_hbm.at[i_vmem.at[0]], o_vmem)

        n_idx = i_hbm.shape[1]
        pltpu.emit_pipeline(
            body,
            grid=(n_idx // GW,),
            in_specs=[pl.BlockSpec((1, GW), index_map=lambda i: (0, i))],
            out_specs=[pl.BlockSpec((GW, f_hbm.shape[1]),
                                    index_map=lambda i: (i, 0))],
            core_axis_name=("core", "subcore"),
            dimension_semantics=(pltpu.PARALLEL,),
        )(i_hbm, g_hbm)


def _interp_kernel(d2_ref, d3_ref, d4_ref, g2_ref, g3_ref, g4_ref,
                   wfc_ref, bcls_ref, out_ref, *, cs):
    ps = []
    for (d_ref, g_ref), c in zip(((d2_ref, g2_ref), (d3_ref, g3_ref),
                                  (d4_ref, g4_ref)), cs):
        dr0 = 1.0 / (d_ref[0, :] + 1e-08)
        dr1 = 1.0 / (d_ref[1, :] + 1e-08)
        dr2 = 1.0 / (d_ref[2, :] + 1e-08)
        norm = jnp.sum(jnp.stack([dr0, dr1, dr2], axis=0), axis=0)
        w = jnp.stack([dr0 / norm, dr1 / norm, dr2 / norm], axis=1)
        gath = jnp.stack([g_ref[0][:, :c], g_ref[1][:, :c],
                          g_ref[2][:, :c]], axis=1)
        ps.append(jnp.sum(gath * w[:, :, None], axis=1))
    pf = jnp.concatenate(ps, axis=1)
    pw = jnp.dot(pf, wfc_ref[...].T, preferred_element_type=jnp.float32)
    out = jnp.dot(pw, bcls_ref[...], preferred_element_type=jnp.float32)
    out_ref[...] = out[:, 0:1]


def kernel(points, xyz2, feat2, xyz3, feat3, xyz4, feat4, W_fc, W_cls):
    n = points.shape[0]
    m2, m3, m4 = xyz2.shape[0], xyz3.shape[0], xyz4.shape[0]
    c2, c3, c4 = feat2.shape[1], feat3.shape[1], feat4.shape[1]
    mt = m2 + m3 + m4

    qt = points.T
    kt = jnp.concatenate([xyz2, xyz3, xyz4], axis=0).T

    c2p = 128
    feat2p = jnp.pad(feat2, ((0, 0), (0, c2p - c2)))
    bcls = jnp.pad(W_cls.T, ((0, 0), (0, 127)))

    nh = n // 2
    sc_gather = pl.kernel(
        _gather_kernel,
        out_type=[
            jax.ShapeDtypeStruct((3 * nh, c2p), jnp.float32),
            jax.ShapeDtypeStruct((3 * nh, c3), jnp.float32),
            jax.ShapeDtypeStruct((3 * nh, c4), jnp.float32),
        ],
        mesh=plsc.VectorSubcoreMesh(core_axis_name="core",
                                    subcore_axis_name="subcore"),
    )

    d_spec = pl.BlockSpec((3, QB), lambda i: (0, i))
    outs = []
    for h in range(2):
        qt_h = jax.lax.slice(qt, (0, h * nh), (3, (h + 1) * nh))
        d2, i2, d3, i3, d4, i4 = pl.pallas_call(
            functools.partial(_nn_kernel, sizes=(m2, m3, m4)),
            grid=(nh // QB,),
            in_specs=[
                pl.BlockSpec((3, QB), lambda i: (0, i)),
                pl.BlockSpec((3, mt), lambda i: (0, 0)),
            ],
            out_specs=[d_spec] * 6,
            out_shape=[
                jax.ShapeDtypeStruct((3, nh), jnp.float32),
                jax.ShapeDtypeStruct((3, nh), jnp.int32),
            ] * 3,
            compiler_params=pltpu.CompilerParams(
                dimension_semantics=("parallel",),
            ),
        )(qt_h, kt)

        g2, g3, g4 = sc_gather(feat2p, i2.reshape(1, 3 * nh),
                               feat3, i3.reshape(1, 3 * nh),
                               feat4, i4.reshape(1, 3 * nh))

        out_h = pl.pallas_call(
            functools.partial(_interp_kernel, cs=(c2, c3, c4)),
            grid=(nh // IB,),
            in_specs=[
                pl.BlockSpec((3, IB), lambda i: (0, i)),
                pl.BlockSpec((3, IB), lambda i: (0, i)),
                pl.BlockSpec((3, IB), lambda i: (0, i)),
                pl.BlockSpec((3, IB, c2p), lambda i: (0, i, 0)),
                pl.BlockSpec((3, IB, c3), lambda i: (0, i, 0)),
                pl.BlockSpec((3, IB, c4), lambda i: (0, i, 0)),
                pl.BlockSpec((64, 320), lambda i: (0, 0)),
                pl.BlockSpec((64, 128), lambda i: (0, 0)),
            ],
            out_specs=pl.BlockSpec((IB, 1), lambda i: (i, 0)),
            out_shape=jax.ShapeDtypeStruct((nh, 1), jnp.float32),
            compiler_params=pltpu.CompilerParams(
                dimension_semantics=("parallel",),
            ),
        )(d2, d3, d4, g2.reshape(3, nh, c2p), g3.reshape(3, nh, c3),
          g4.reshape(3, nh, c4), W_fc, bcls)
        outs.append(out_h)

    return jnp.concatenate(outs, axis=0)


# speedup vs baseline: 16.3678x; 1.1020x over previous
import functools

import jax
import jax.numpy as jnp
from jax.experimental import pallas as pl
from jax.experimental.pallas import tpu as pltpu
from jax.experimental.pallas import tpu_sc as plsc

QB = 128
IB = 1024
GW = 128


def _top3(d_tiles, m):
    qb = d_tiles[0].shape[0]
    inf = jnp.full((qb, 128), jnp.inf, jnp.float32)
    zero = jnp.zeros((qb, 128), jnp.int32)
    b1, b2, b3 = inf, inf, inf
    t1, t2, t3 = zero, zero, zero
    for t, v in enumerate(d_tiles):
        c1 = v < b1
        c2 = v < b2
        c3 = v < b3
        nb3 = jnp.where(c2, b2, jnp.where(c3, v, b3))
        nt3 = jnp.where(c2, t2, jnp.where(c3, t, t3))
        nb2 = jnp.where(c1, b1, jnp.where(c2, v, b2))
        nt2 = jnp.where(c1, t1, jnp.where(c2, t, t2))
        b1 = jnp.where(c1, v, b1)
        t1 = jnp.where(c1, t, t1)
        b2, b3, t2, t3 = nb2, nb3, nt2, nt3
    lane = jax.lax.broadcasted_iota(jnp.int32, (qb, 128), 1)
    i1 = t1 * 128 + lane
    i2 = t2 * 128 + lane
    i3 = t3 * 128 + lane
    dists = []
    idxs = []
    for k in range(3):
        mv = jnp.min(b1, axis=1)
        hitv = b1 == mv[:, None]
        fi = jnp.min(jnp.where(hitv, i1, m), axis=1)
        dists.append(mv)
        idxs.append(fi)
        if k < 2:
            hit = hitv & (i1 == fi[:, None])
            b1 = jnp.where(hit, b2, b1)
            i1 = jnp.where(hit, i2, i1)
            b2 = jnp.where(hit, b3, b2)
            i2 = jnp.where(hit, i3, i2)
            b3 = jnp.where(hit, jnp.float32(jnp.inf), b3)
    return dists, idxs


def _nn_kernel(qt_ref, kt_ref, d2_ref, i2_ref, d3_ref, i3_ref, d4_ref,
               i4_ref, *, sizes):
    qt = qt_ref[...]
    kt = kt_ref[...]
    qq = jnp.sum(qt * qt, axis=0)
    kk = jnp.sum(kt * kt, axis=0)
    dot = jnp.dot(qt.T, kt, preferred_element_type=jnp.float32)
    off = 0
    for (d_ref, i_ref), m in zip(((d2_ref, i2_ref), (d3_ref, i3_ref),
                                  (d4_ref, i4_ref)), sizes):
        tiles = [
            qq[:, None] + kk[None, off + t * 128:off + (t + 1) * 128]
            - 2.0 * dot[:, off + t * 128:off + (t + 1) * 128]
            for t in range(m // 128)
        ]
        dists, idxs = _top3(tiles, m)
        for k in range(3):
            d_ref[k, :] = dists[k]
            i_ref[k, :] = idxs[k]
        off += m


def _gather_kernel(f2_hbm, i2_hbm, f3_hbm, i3_hbm, f4_hbm, i4_hbm,
                   g2_hbm, g3_hbm, g4_hbm):
    for f_hbm, i_hbm, g_hbm in ((f2_hbm, i2_hbm, g2_hbm),
                                (f3_hbm, i3_hbm, g3_hbm),
                                (f4_hbm, i4_hbm, g4_hbm)):
        def body(i_vmem, o_vmem, f_hbm=f_hbm):
            pltpu.sync_copy(f_hbm.at[i_vmem.at[0]], o_vmem)

        n_idx = i_hbm.shape[1]
        pltpu.emit_pipeline(
            body,
            grid=(n_idx // GW,),
            in_specs=[pl.BlockSpec((1, GW), index_map=lambda i: (0, i))],
            out_specs=[pl.BlockSpec((GW, f_hbm.shape[1]),
                                    index_map=lambda i: (i, 0))],
            core_axis_name=("core", "subcore"),
            dimension_semantics=(pltpu.PARALLEL,),
        )(i_hbm, g_hbm)


def _interp_kernel(d2_ref, d3_ref, d4_ref, g2_ref, g3_ref, g4_ref,
                   wfc_ref, bcls_ref, out_ref, *, cs):
    ps = []
    for (d_ref, g_ref), c in zip(((d2_ref, g2_ref), (d3_ref, g3_ref),
                                  (d4_ref, g4_ref)), cs):
        dr0 = 1.0 / (d_ref[0, :] + 1e-08)
        dr1 = 1.0 / (d_ref[1, :] + 1e-08)
        dr2 = 1.0 / (d_ref[2, :] + 1e-08)
        norm = jnp.sum(jnp.stack([dr0, dr1, dr2], axis=0), axis=0)
        e0 = g_ref[0][:, :c] * (dr0 / norm)[:, None]
        e1 = g_ref[1][:, :c] * (dr1 / norm)[:, None]
        e2 = g_ref[2][:, :c] * (dr2 / norm)[:, None]
        ps.append((e0 + e1) + e2)
    pf = jnp.concatenate(ps, axis=1)
    pw = jnp.dot(pf, wfc_ref[...].T, preferred_element_type=jnp.float32)
    out = jnp.dot(pw, bcls_ref[...], preferred_element_type=jnp.float32)
    out_ref[...] = out[:, 0:1]


def kernel(points, xyz2, feat2, xyz3, feat3, xyz4, feat4, W_fc, W_cls):
    n = points.shape[0]
    m2, m3, m4 = xyz2.shape[0], xyz3.shape[0], xyz4.shape[0]
    c2, c3, c4 = feat2.shape[1], feat3.shape[1], feat4.shape[1]
    mt = m2 + m3 + m4

    qt = points.T
    kt = jnp.concatenate([xyz2, xyz3, xyz4], axis=0).T

    c2p = 128
    feat2p = jnp.pad(feat2, ((0, 0), (0, c2p - c2)))
    bcls = jnp.pad(W_cls.T, ((0, 0), (0, 127)))

    nh = n // 2
    sc_gather = pl.kernel(
        _gather_kernel,
        out_type=[
            jax.ShapeDtypeStruct((3 * nh, c2p), jnp.float32),
            jax.ShapeDtypeStruct((3 * nh, c3), jnp.float32),
            jax.ShapeDtypeStruct((3 * nh, c4), jnp.float32),
        ],
        mesh=plsc.VectorSubcoreMesh(core_axis_name="core",
                                    subcore_axis_name="subcore"),
    )

    d_spec = pl.BlockSpec((3, QB), lambda i: (0, i))
    outs = []
    for h in range(2):
        qt_h = jax.lax.slice(qt, (0, h * nh), (3, (h + 1) * nh))
        d2, i2, d3, i3, d4, i4 = pl.pallas_call(
            functools.partial(_nn_kernel, sizes=(m2, m3, m4)),
            grid=(nh // QB,),
            in_specs=[
                pl.BlockSpec((3, QB), lambda i: (0, i)),
                pl.BlockSpec((3, mt), lambda i: (0, 0)),
            ],
            out_specs=[d_spec] * 6,
            out_shape=[
                jax.ShapeDtypeStruct((3, nh), jnp.float32),
                jax.ShapeDtypeStruct((3, nh), jnp.int32),
            ] * 3,
            compiler_params=pltpu.CompilerParams(
                dimension_semantics=("parallel",),
            ),
        )(qt_h, kt)

        g2, g3, g4 = sc_gather(feat2p, i2.reshape(1, 3 * nh),
                               feat3, i3.reshape(1, 3 * nh),
                               feat4, i4.reshape(1, 3 * nh))

        out_h = pl.pallas_call(
            functools.partial(_interp_kernel, cs=(c2, c3, c4)),
            grid=(nh // IB,),
            in_specs=[
                pl.BlockSpec((3, IB), lambda i: (0, i)),
                pl.BlockSpec((3, IB), lambda i: (0, i)),
                pl.BlockSpec((3, IB), lambda i: (0, i)),
                pl.BlockSpec((3, IB, c2p), lambda i: (0, i, 0)),
                pl.BlockSpec((3, IB, c3), lambda i: (0, i, 0)),
                pl.BlockSpec((3, IB, c4), lambda i: (0, i, 0)),
                pl.BlockSpec((64, 320), lambda i: (0, 0)),
                pl.BlockSpec((64, 128), lambda i: (0, 0)),
            ],
            out_specs=pl.BlockSpec((IB, 1), lambda i: (i, 0)),
            out_shape=jax.ShapeDtypeStruct((nh, 1), jnp.float32),
            compiler_params=pltpu.CompilerParams(
                dimension_semantics=("parallel",),
            ),
        )(d2, d3, d4, g2.reshape(3, nh, c2p), g3.reshape(3, nh, c3),
          g4.reshape(3, nh, c4), W_fc, bcls)
        outs.append(out_h)

    return jnp.concatenate(outs, axis=0)
